# baseline (speedup 1.0000x reference)
import functools

import jax
import jax.numpy as jnp
from jax import lax
from jax.experimental import pallas as pl
from jax.experimental.pallas import tpu as pltpu
from jax.experimental.pallas import tpu_sc as plsc

_M = 256
_N = 50000

_BP = 1040
_CH = 40
_N_TC = 38480

_NC, _NS, _L = 2, 16, 16
_NW = _NC * _NS
_P_SC = _N - _N_TC
_P_TILE = _P_SC // _NW
_CP = 72
_NCH = _P_TILE // _CP
_NG = _M // _L


def _tc_chunk_sums(x_raw, t):
    s = jax.nn.sigmoid(x_raw)
    return s * t, s, t


def _tc_body(inp_ref, tgt_ref, out_ref, a_st_ref, a_s_ref, a_t_ref):
    i = pl.program_id(0)
    n_steps = pl.num_programs(0)
    bp, m = inp_ref.shape

    z = jnp.zeros((_CH, m), jnp.float32)
    a_st, a_s, a_t = z, z, z
    for k in range(bp // _CH):
        st, s, t = _tc_chunk_sums(
            inp_ref[k * _CH:(k + 1) * _CH, :], tgt_ref[k * _CH:(k + 1) * _CH, :]
        )
        a_st, a_s, a_t = a_st + st, a_s + s, a_t + t

    @pl.when(i == 0)
    def _init():
        a_st_ref[...] = a_st
        a_s_ref[...] = a_s
        a_t_ref[...] = a_t

    @pl.when(i > 0)
    def _accum():
        a_st_ref[...] += a_st
        a_s_ref[...] += a_s
        a_t_ref[...] += a_t

    @pl.when(i == n_steps - 1)
    def _final():
        out_ref[0:1, :] = jnp.sum(a_st_ref[...], axis=0, keepdims=True)
        out_ref[1:2, :] = jnp.sum(a_s_ref[...], axis=0, keepdims=True)
        out_ref[2:3, :] = jnp.sum(a_t_ref[...], axis=0, keepdims=True)


def _tc_partial(xt, tt):
    return pl.pallas_call(
        _tc_body,
        grid=(_N_TC // _BP,),
        in_specs=[
            pl.BlockSpec((_BP, _M), lambda i: (i, 0)),
            pl.BlockSpec((_BP, _M), lambda i: (i, 0)),
        ],
        out_specs=pl.BlockSpec((8, _M), lambda i: (0, 0)),
        out_shape=jax.ShapeDtypeStruct((8, _M), jnp.float32),
        scratch_shapes=[pltpu.VMEM((_CH, _M), jnp.float32)] * 3,
    )(xt, tt)


def _make_sc_partial():
    mesh = plsc.VectorSubcoreMesh(core_axis_name="c", subcore_axis_name="s")

    @functools.partial(
        pl.kernel,
        mesh=mesh,
        out_type=jax.ShapeDtypeStruct((_NW, 8, _M), jnp.float32),
        scratch_types=[
            pltpu.VMEM((_CP, _M), jnp.float32),
            pltpu.VMEM((_CP, _M), jnp.float32),
            pltpu.VMEM((8, _M), jnp.float32),
        ],
    )
    def sc_partial(x_hbm, t_hbm, out_hbm, xv, tv, acc):
        wid = lax.axis_index("s") * _NC + lax.axis_index("c")
        p0 = _N_TC + wid * _P_TILE

        def chunk(ci, _):
            pltpu.sync_copy(x_hbm.at[pl.ds(p0 + ci * _CP, _CP)], xv)
            pltpu.sync_copy(t_hbm.at[pl.ds(p0 + ci * _CP, _CP)], tv)
            for g in range(_NG):
                sl = pl.ds(g * _L, _L)

                def body(p, carry):
                    a_st, a_s, a_t = carry
                    x = xv[p, sl]
                    t = tv[p, sl]
                    s = 1.0 / (1.0 + jnp.exp(-x))
                    return (a_st + s * t, a_s + s, a_t + t)

                zv = jnp.zeros((_L,), jnp.float32)
                a_st, a_s, a_t = lax.fori_loop(0, _CP, body, (zv, zv, zv))

                @pl.when(ci == 0)
                def _init():
                    acc[0, sl] = a_st
                    acc[1, sl] = a_s
                    acc[2, sl] = a_t

                @pl.when(ci > 0)
                def _acc():
                    acc[0, sl] += a_st
                    acc[1, sl] += a_s
                    acc[2, sl] += a_t
            return 0

        lax.fori_loop(0, _NCH, chunk, 0)
        pltpu.sync_copy(acc, out_hbm.at[wid])

    return sc_partial


_sc_partial = _make_sc_partial()


def _combine_body(tc_ref, sc_ref, out_ref):
    part = tc_ref[...] + jnp.sum(sc_ref[...], axis=0)
    sum_st = part[0:1, :]
    sum_s = part[1:2, :]
    sum_t = part[2:3, :]
    loss = 1.0 - (2.0 * sum_st + 1.0) / (sum_s + sum_t + 1.0)
    out_ref[...] = jnp.sum(loss).reshape(1, 1)


def _combine(tc_part, sc_part):
    return pl.pallas_call(
        _combine_body,
        out_shape=jax.ShapeDtypeStruct((1, 1), jnp.float32),
    )(tc_part, sc_part)


def kernel(inputs, targets, num_boxes):
    xt = inputs.T
    tt = targets.T
    tc_part = _tc_partial(xt, tt)
    sc_part = _sc_partial(xt, tt)
    total = _combine(tc_part, sc_part)
    return total[0, 0] / (num_boxes + 1e-6)


# speedup vs baseline: 1.0849x; 1.0849x over previous
import functools

import jax
import jax.numpy as jnp
from jax import lax
from jax.experimental import pallas as pl
from jax.experimental.pallas import tpu as pltpu
from jax.experimental.pallas import tpu_sc as plsc

_M = 256
_N = 50000

_BP = 2960
_CH = 40
_N_TC = 38480

_NC, _NS, _L = 2, 16, 16
_NW = _NC * _NS
_P_SC = _N - _N_TC
_P_TILE = _P_SC // _NW
_CP = 72
_NCH = _P_TILE // _CP
_NG = _M // _L
_UP = 8


def _tc_chunk_sums(x_raw, t):
    s = jax.nn.sigmoid(x_raw)
    return s * t, s, t


def _tc_body(inp_ref, tgt_ref, out_ref, a_st_ref, a_s_ref, a_t_ref):
    i = pl.program_id(0)
    n_steps = pl.num_programs(0)
    bp, m = inp_ref.shape

    z = jnp.zeros((_CH, m), jnp.float32)
    a_st, a_s, a_t = z, z, z
    for k in range(bp // _CH):
        st, s, t = _tc_chunk_sums(
            inp_ref[k * _CH:(k + 1) * _CH, :], tgt_ref[k * _CH:(k + 1) * _CH, :]
        )
        a_st, a_s, a_t = a_st + st, a_s + s, a_t + t

    @pl.when(i == 0)
    def _init():
        a_st_ref[...] = a_st
        a_s_ref[...] = a_s
        a_t_ref[...] = a_t

    @pl.when(i > 0)
    def _accum():
        a_st_ref[...] += a_st
        a_s_ref[...] += a_s
        a_t_ref[...] += a_t

    @pl.when(i == n_steps - 1)
    def _final():
        out_ref[0:1, :] = jnp.sum(a_st_ref[...], axis=0, keepdims=True)
        out_ref[1:2, :] = jnp.sum(a_s_ref[...], axis=0, keepdims=True)
        out_ref[2:3, :] = jnp.sum(a_t_ref[...], axis=0, keepdims=True)


def _tc_partial(xt, tt):
    return pl.pallas_call(
        _tc_body,
        grid=(_N_TC // _BP,),
        in_specs=[
            pl.BlockSpec((_BP, _M), lambda i: (i, 0)),
            pl.BlockSpec((_BP, _M), lambda i: (i, 0)),
        ],
        out_specs=pl.BlockSpec((8, _M), lambda i: (0, 0)),
        out_shape=jax.ShapeDtypeStruct((8, _M), jnp.float32),
        scratch_shapes=[pltpu.VMEM((_CH, _M), jnp.float32)] * 3,
    )(xt, tt)


def _make_sc_partial():
    mesh = plsc.VectorSubcoreMesh(core_axis_name="c", subcore_axis_name="s")

    @functools.partial(
        pl.kernel,
        mesh=mesh,
        out_type=jax.ShapeDtypeStruct((_NW, 8, _M), jnp.float32),
        scratch_types=[
            pltpu.VMEM((_CP, _M), jnp.float32),
            pltpu.VMEM((_CP, _M), jnp.float32),
            pltpu.VMEM((8, _M), jnp.float32),
        ],
    )
    def sc_partial(x_hbm, t_hbm, out_hbm, xv, tv, acc):
        wid = lax.axis_index("s") * _NC + lax.axis_index("c")
        p0 = _N_TC + wid * _P_TILE

        def chunk(ci, _):
            pltpu.sync_copy(x_hbm.at[pl.ds(p0 + ci * _CP, _CP)], xv)
            pltpu.sync_copy(t_hbm.at[pl.ds(p0 + ci * _CP, _CP)], tv)
            for g in range(_NG):
                sl = pl.ds(g * _L, _L)

                def body(q, carry):
                    a_st, a_s, a_t = carry
                    for r in range(_UP):
                        p = q * _UP + r
                        x = xv[p, sl]
                        t = tv[p, sl]
                        s = 1.0 / (1.0 + jnp.exp(-x))
                        a_st, a_s, a_t = a_st + s * t, a_s + s, a_t + t
                    return (a_st, a_s, a_t)

                zv = jnp.zeros((_L,), jnp.float32)
                a_st, a_s, a_t = lax.fori_loop(0, _CP // _UP, body, (zv, zv, zv))

                @pl.when(ci == 0)
                def _init():
                    acc[0, sl] = a_st
                    acc[1, sl] = a_s
                    acc[2, sl] = a_t

                @pl.when(ci > 0)
                def _acc():
                    acc[0, sl] += a_st
                    acc[1, sl] += a_s
                    acc[2, sl] += a_t
            return 0

        lax.fori_loop(0, _NCH, chunk, 0)
        pltpu.sync_copy(acc, out_hbm.at[wid])

    return sc_partial


_sc_partial = _make_sc_partial()


def _combine_body(tc_ref, sc_ref, out_ref):
    part = tc_ref[...] + jnp.sum(sc_ref[...], axis=0)
    sum_st = part[0:1, :]
    sum_s = part[1:2, :]
    sum_t = part[2:3, :]
    loss = 1.0 - (2.0 * sum_st + 1.0) / (sum_s + sum_t + 1.0)
    out_ref[...] = jnp.sum(loss).reshape(1, 1)


def _combine(tc_part, sc_part):
    return pl.pallas_call(
        _combine_body,
        out_shape=jax.ShapeDtypeStruct((1, 1), jnp.float32),
    )(tc_part, sc_part)


def kernel(inputs, targets, num_boxes):
    xt = inputs.T
    tt = targets.T
    tc_part = _tc_partial(xt, tt)
    sc_part = _sc_partial(xt, tt)
    total = _combine(tc_part, sc_part)
    return total[0, 0] / (num_boxes + 1e-6)


# speedup vs baseline: 1.2592x; 1.1607x over previous
import functools

import jax
import jax.numpy as jnp
from jax import lax
from jax.experimental import pallas as pl
from jax.experimental.pallas import tpu as pltpu
from jax.experimental.pallas import tpu_sc as plsc

_M = 256
_N = 50000

_BP = 2960
_CH = 40
_N_TC = 38480

_NC, _NS, _L = 2, 16, 16
_NW = _NC * _NS
_P_SC = _N - _N_TC
_P_TILE = _P_SC // _NW
_CP = 40
_NCH = _P_TILE // _CP
_NG = _M // _L
_UP = 8


def _tc_chunk_sums(x_raw, t):
    s = jax.nn.sigmoid(x_raw)
    return s * t, s, t


def _tc_body(inp_ref, tgt_ref, out_ref, a_st_ref, a_s_ref, a_t_ref):
    i = pl.program_id(0)
    n_steps = pl.num_programs(0)
    bp, m = inp_ref.shape

    z = jnp.zeros((_CH, m), jnp.float32)
    a_st, a_s, a_t = z, z, z
    for k in range(bp // _CH):
        st, s, t = _tc_chunk_sums(
            inp_ref[k * _CH:(k + 1) * _CH, :], tgt_ref[k * _CH:(k + 1) * _CH, :]
        )
        a_st, a_s, a_t = a_st + st, a_s + s, a_t + t

    @pl.when(i == 0)
    def _init():
        a_st_ref[...] = a_st
        a_s_ref[...] = a_s
        a_t_ref[...] = a_t

    @pl.when(i > 0)
    def _accum():
        a_st_ref[...] += a_st
        a_s_ref[...] += a_s
        a_t_ref[...] += a_t

    @pl.when(i == n_steps - 1)
    def _final():
        out_ref[0:1, :] = jnp.sum(a_st_ref[...], axis=0, keepdims=True)
        out_ref[1:2, :] = jnp.sum(a_s_ref[...], axis=0, keepdims=True)
        out_ref[2:3, :] = jnp.sum(a_t_ref[...], axis=0, keepdims=True)


def _tc_partial(xt, tt):
    return pl.pallas_call(
        _tc_body,
        grid=(_N_TC // _BP,),
        in_specs=[
            pl.BlockSpec((_BP, _M), lambda i: (i, 0)),
            pl.BlockSpec((_BP, _M), lambda i: (i, 0)),
        ],
        out_specs=pl.BlockSpec((8, _M), lambda i: (0, 0)),
        out_shape=jax.ShapeDtypeStruct((8, _M), jnp.float32),
        scratch_shapes=[pltpu.VMEM((_CH, _M), jnp.float32)] * 3,
    )(xt, tt)


def _make_sc_partial():
    mesh = plsc.VectorSubcoreMesh(core_axis_name="c", subcore_axis_name="s")

    @functools.partial(
        pl.kernel,
        mesh=mesh,
        out_type=jax.ShapeDtypeStruct((_NW, 8,_M), jnp.float32),
        scratch_types=[
            pltpu.VMEM((_CP,_M), jnp.float32),
            pltpu.VMEM((_CP,_M), jnp.float32),
            pltpu.VMEM((_CP,_M), jnp.float32),
            pltpu.VMEM((_CP,_M), jnp.float32),
            pltpu.VMEM((8,_M), jnp.float32),
            pltpu.SemaphoreType.DMA,
            pltpu.SemaphoreType.DMA,
            pltpu.SemaphoreType.DMA,
            pltpu.SemaphoreType.DMA,
        ],
    )
    def sc_partial(x_hbm, t_hbm, out_hbm, xv0, tv0, xv1, tv1, acc,
                   sx0, st0, sx1, st1):
        wid = lax.axis_index("s") * _NC + lax.axis_index("c")
        p0 = _N_TC + wid * _P_TILE
        bufs = ((xv0, tv0, sx0, st0), (xv1, tv1, sx1, st1))

        def start(ci, b):
            xv, tv, sx, st = b
            rows = pl.ds(p0 + ci * _CP, _CP)
            pltpu.make_async_copy(x_hbm.at[rows], xv, sx).start()
            pltpu.make_async_copy(t_hbm.at[rows], tv, st).start()

        def wait(ci, b):
            xv, tv, sx, st = b
            rows = pl.ds(p0 + ci * _CP, _CP)
            pltpu.make_async_copy(x_hbm.at[rows], xv, sx).wait()
            pltpu.make_async_copy(t_hbm.at[rows], tv, st).wait()

        start(0, bufs[0])
        for ci in range(_NCH):
            b = bufs[ci % 2]
            wait(ci, b)
            if ci + 1 < _NCH:
                start(ci + 1, bufs[(ci + 1) % 2])
            xv, tv = b[0], b[1]
            first = ci == 0

            def g_body(g, _):
                sl = pl.ds(g *_L,_L)

                def body(q, carry):
                    a_st, a_s, a_t = carry
                    for r in range(_UP):
                        p = q * _UP + r
                        x = xv[p, sl]
                        t = tv[p, sl]
                        s = 1.0 / (1.0 + jnp.exp(-x))
                        a_st, a_s, a_t = a_st + s * t, a_s + s, a_t + t
                    return (a_st, a_s, a_t)

                zv = jnp.zeros((_L,), jnp.float32)
                a_st, a_s, a_t = lax.fori_loop(0, _CP // _UP, body, (zv, zv, zv))
                if first:
                    acc[0, sl] = a_st
                    acc[1, sl] = a_s
                    acc[2, sl] = a_t
                else:
                    acc[0, sl] += a_st
                    acc[1, sl] += a_s
                    acc[2, sl] += a_t
                return 0

            lax.fori_loop(0, _NG, g_body, 0)
        pltpu.sync_copy(acc, out_hbm.at[wid])

    return sc_partial


_sc_partial = _make_sc_partial()


def _combine_body(tc_ref, sc_ref, out_ref):
    part = tc_ref[...] + jnp.sum(sc_ref[...], axis=0)
    sum_st = part[0:1, :]
    sum_s = part[1:2, :]
    sum_t = part[2:3, :]
    loss = 1.0 - (2.0 * sum_st + 1.0) / (sum_s + sum_t + 1.0)
    out_ref[...] = jnp.sum(loss).reshape(1, 1)


def _combine(tc_part, sc_part):
    return pl.pallas_call(
        _combine_body,
        out_shape=jax.ShapeDtypeStruct((1, 1), jnp.float32),
    )(tc_part, sc_part)


def kernel(inputs, targets, num_boxes):
    xt = inputs.T
    tt = targets.T
    tc_part = _tc_partial(xt, tt)
    sc_part = _sc_partial(xt, tt)
    total = _combine(tc_part, sc_part)
    return total[0, 0] / (num_boxes + 1e-6)


# speedup vs baseline: 1.9288x; 1.5318x over previous
import jax
import jax.numpy as jnp
from jax.experimental import pallas as pl
from jax.experimental.pallas import tpu as pltpu

_BP = 5000
_CH = 40

_C = (
    0.24990395925961004,
    -0.020435871793313163,
    0.001795901034182633,
    -0.00012303520659997033,
    5.729155408298089e-06,
    -1.649533378409172e-07,
    2.6158928545591356e-09,
    -1.7372812469973818e-11,
)


def _chunk_sums(x_raw, t):
    s = jax.nn.sigmoid(x_raw)
    return s * t, s, t


def _dice_body(inp_ref, tgt_ref, out_ref, a_st_ref, a_s_ref, a_t_ref):
    i = pl.program_id(0)
    n_steps = pl.num_programs(0)
    bp, m = inp_ref.shape

    z = jnp.zeros((_CH, m), jnp.float32)
    a_st, a_s, a_t = z, z, z
    for k in range(bp // _CH):
        st, s, t = _chunk_sums(
            inp_ref[k * _CH:(k + 1) * _CH, :], tgt_ref[k * _CH:(k + 1) * _CH, :]
        )
        a_st, a_s, a_t = a_st + st, a_s + s, a_t + t

    @pl.when(i == 0)
    def _init():
        a_st_ref[...] = a_st
        a_s_ref[...] = a_s
        a_t_ref[...] = a_t

    @pl.when(i > 0)
    def _accum():
        a_st_ref[...] += a_st
        a_s_ref[...] += a_s
        a_t_ref[...] += a_t

    @pl.when(i == n_steps - 1)
    def _final():
        sum_st = jnp.sum(a_st_ref[...], axis=0)
        sum_s = jnp.sum(a_s_ref[...], axis=0)
        sum_t = jnp.sum(a_t_ref[...], axis=0)
        num = 2.0 * sum_st
        den = sum_s + sum_t
        loss = 1.0 - (num + 1.0) / (den + 1.0)
        out_ref[...] = jnp.sum(loss).reshape(1, 1)


def kernel(inputs, targets, num_boxes):
    n_masks, n_points = inputs.shape
    xt = inputs.T
    tt = targets.T
    total = pl.pallas_call(
        _dice_body,
        grid=(n_points // _BP,),
        in_specs=[
            pl.BlockSpec((_BP, n_masks), lambda i: (i, 0)),
            pl.BlockSpec((_BP, n_masks), lambda i: (i, 0)),
        ],
        out_specs=pl.BlockSpec((1, 1), lambda i: (0, 0)),
        out_shape=jax.ShapeDtypeStruct((1, 1), jnp.float32),
        scratch_shapes=[pltpu.VMEM((_CH, n_masks), jnp.float32)] * 3,
    )(xt, tt)
    return total[0, 0] / (num_boxes + 1e-6)


# speedup vs baseline: 1.9326x; 1.0020x over previous
import jax
import jax.numpy as jnp
from jax.experimental import pallas as pl
from jax.experimental.pallas import tpu as pltpu

_BP = 5000
_CH = 40


def _chunk_sums(x_raw, t):
    s = jax.nn.sigmoid(x_raw)
    return s * t, s, t


def _dice_body(inp_ref, tgt_ref, out_ref, a_st_ref, a_s_ref, a_t_ref):
    i = pl.program_id(0)
    n_steps = pl.num_programs(0)
    bp, m = inp_ref.shape

    z = jnp.zeros((_CH, m), jnp.float32)
    a_st, a_s, a_t = z, z, z
    for k in range(bp // _CH):
        st, s, t = _chunk_sums(
            inp_ref[k * _CH:(k + 1) * _CH, :], tgt_ref[k * _CH:(k + 1) * _CH, :]
        )
        a_st, a_s, a_t = a_st + st, a_s + s, a_t + t

    @pl.when(i == 0)
    def _init():
        a_st_ref[...] = a_st
        a_s_ref[...] = a_s
        a_t_ref[...] = a_t

    @pl.when(i > 0)
    def _accum():
        a_st_ref[...] += a_st
        a_s_ref[...] += a_s
        a_t_ref[...] += a_t

    @pl.when(i == n_steps - 1)
    def _final():
        sum_st = jnp.sum(a_st_ref[...], axis=0)
        sum_s = jnp.sum(a_s_ref[...], axis=0)
        sum_t = jnp.sum(a_t_ref[...], axis=0)
        num = 2.0 * sum_st
        den = sum_s + sum_t
        loss = 1.0 - (num + 1.0) / (den + 1.0)
        out_ref[...] = jnp.sum(loss).reshape(1, 1)


def kernel(inputs, targets, num_boxes):
    n_masks, n_points = inputs.shape
    xt = inputs.T
    tt = targets.T
    total = pl.pallas_call(
        _dice_body,
        grid=(n_points // _BP,),
        in_specs=[
            pl.BlockSpec((_BP, n_masks), lambda i: (i, 0)),
            pl.BlockSpec((_BP, n_masks), lambda i: (i, 0)),
        ],
        out_specs=pl.BlockSpec((1, 1), lambda i: (0, 0)),
        out_shape=jax.ShapeDtypeStruct((1, 1), jnp.float32),
        scratch_shapes=[pltpu.VMEM((_CH, n_masks), jnp.float32)] * 3,
    )(xt, tt)
    return total[0, 0] / (num_boxes + 1e-6)
